# baseline (device time: 161684 ns/iter reference)
import jax
import jax.numpy as jnp
from jax import lax
from jax.experimental import pallas as pl
from jax.experimental.pallas import tpu as pltpu

N_DEV = 8
B, SQ, SKV, HQ, DH = 2, 256, 2048, 32, 64
H_LOC = HQ // N_DEV
SKV_LOC = SKV // N_DEV
HD_LOC = H_LOC * DH
D_MODEL = 512


def kernel(x, Wq, K_ext, V_ext, Wo):
    def body(x_ref, wq_ref, k_ref, v_ref, wo_ref, out_ref,
             ksend, vsend, khbuf, vhbuf, pbuf, ctx2,
             ksend_sems, krecv_sems, vsend_sems, vrecv_sems,
             psend_sems, precv_sems):
        my = lax.axis_index("i")

        bar = pltpu.get_barrier_semaphore()
        for k in range(1, N_DEV):
            peer = lax.rem(my + k, N_DEV)
            pl.semaphore_signal(bar, inc=1, device_id=(peer,),
                                device_id_type=pl.DeviceIdType.MESH)
        pl.semaphore_wait(bar, N_DEV - 1)

        for i in range(N_DEV):
            ksend[i] = k_ref[:, :, H_LOC * i:H_LOC * (i + 1), :].reshape(
                B, SKV_LOC, HD_LOC)
            vsend[i] = v_ref[:, :, H_LOC * i:H_LOC * (i + 1), :].reshape(
                B, SKV_LOC, HD_LOC)

        khbuf[my] = ksend[my]
        vhbuf[my] = vsend[my]

        for k in range(1, N_DEV):
            dst = lax.rem(my + k, N_DEV)
            pltpu.make_async_remote_copy(
                src_ref=ksend.at[dst], dst_ref=khbuf.at[my],
                send_sem=ksend_sems.at[dst], recv_sem=krecv_sems.at[my],
                device_id=(dst,), device_id_type=pl.DeviceIdType.MESH,
            ).start()
            pltpu.make_async_remote_copy(
                src_ref=vsend.at[dst], dst_ref=vhbuf.at[my],
                send_sem=vsend_sems.at[dst], recv_sem=vrecv_sems.at[my],
                device_id=(dst,), device_id_type=pl.DeviceIdType.MESH,
            ).start()

        q2 = jnp.dot(x_ref[...].reshape(B * SQ, D_MODEL), wq_ref[...],
                     preferred_element_type=jnp.float32)

        qb = lax.broadcasted_iota(jnp.int32, (SQ, SKV), 0) // 64
        kb = lax.broadcasted_iota(jnp.int32, (SQ, SKV), 1) // 64
        mask = (qb == kb) | (kb == 0) | ((qb + kb) % 3 == 0)

        for k in range(1, N_DEV):
            src = lax.rem(my + k, N_DEV)
            pltpu.make_async_remote_copy(
                src_ref=ksend.at[0], dst_ref=khbuf.at[src],
                send_sem=ksend_sems.at[0], recv_sem=krecv_sems.at[src],
                device_id=(src,), device_id_type=pl.DeviceIdType.MESH,
            ).wait_recv()
            pltpu.make_async_remote_copy(
                src_ref=vsend.at[0], dst_ref=vhbuf.at[src],
                send_sem=vsend_sems.at[0], recv_sem=vrecv_sems.at[src],
                device_id=(src,), device_id_type=pl.DeviceIdType.MESH,
            ).wait_recv()

        for b in range(B):
            for hl in range(H_LOC):
                q_bh = q2[b * SQ:(b + 1) * SQ, hl * DH:(hl + 1) * DH]
                kf = khbuf[:, b, :, hl * DH:(hl + 1) * DH].reshape(SKV, DH)
                s = lax.dot_general(
                    q_bh, kf, (((1,), (1,)), ((), ())),
                    preferred_element_type=jnp.float32) * 0.125
                s = jnp.where(mask, s, -1e9)
                m = jnp.max(s, axis=1, keepdims=True)
                w = jnp.exp(s - m)
                w = w / jnp.sum(w, axis=1, keepdims=True)
                vf = vhbuf[:, b, :, hl * DH:(hl + 1) * DH].reshape(SKV, DH)
                ctx2[b * SQ:(b + 1) * SQ, hl * DH:(hl + 1) * DH] = jnp.dot(
                    w, vf, preferred_element_type=jnp.float32)

        pbuf[my] = jnp.dot(ctx2[...], wo_ref[...],
                           preferred_element_type=jnp.float32)
        for k in range(1, N_DEV):
            dst = lax.rem(my + k, N_DEV)
            pltpu.make_async_remote_copy(
                src_ref=pbuf.at[my], dst_ref=pbuf.at[my],
                send_sem=psend_sems.at[dst], recv_sem=precv_sems.at[my],
                device_id=(dst,), device_id_type=pl.DeviceIdType.MESH,
            ).start()
        for k in range(1, N_DEV):
            src = lax.rem(my + k, N_DEV)
            pltpu.make_async_remote_copy(
                src_ref=pbuf.at[0], dst_ref=pbuf.at[src],
                send_sem=psend_sems.at[0], recv_sem=precv_sems.at[src],
                device_id=(src,), device_id_type=pl.DeviceIdType.MESH,
            ).wait_recv()

        out_ref[...] = jnp.sum(pbuf[...], axis=0).reshape(B, SQ, D_MODEL)

        for k in range(1, N_DEV):
            dst = lax.rem(my + k, N_DEV)
            pltpu.make_async_remote_copy(
                src_ref=ksend.at[dst], dst_ref=khbuf.at[0],
                send_sem=ksend_sems.at[dst], recv_sem=krecv_sems.at[0],
                device_id=(dst,), device_id_type=pl.DeviceIdType.MESH,
            ).wait_send()
            pltpu.make_async_remote_copy(
                src_ref=vsend.at[dst], dst_ref=vhbuf.at[0],
                send_sem=vsend_sems.at[dst], recv_sem=vrecv_sems.at[0],
                device_id=(dst,), device_id_type=pl.DeviceIdType.MESH,
            ).wait_send()
            pltpu.make_async_remote_copy(
                src_ref=pbuf.at[my], dst_ref=pbuf.at[0],
                send_sem=psend_sems.at[dst], recv_sem=precv_sems.at[0],
                device_id=(dst,), device_id_type=pl.DeviceIdType.MESH,
            ).wait_send()

    return pl.pallas_call(
        body,
        out_shape=jax.ShapeDtypeStruct((B, SQ, D_MODEL), jnp.float32),
        in_specs=[pl.BlockSpec(memory_space=pltpu.VMEM)] * 5,
        out_specs=pl.BlockSpec(memory_space=pltpu.VMEM),
        scratch_shapes=[
            pltpu.VMEM((N_DEV, B, SKV_LOC, HD_LOC), jnp.float32),
            pltpu.VMEM((N_DEV, B, SKV_LOC, HD_LOC), jnp.float32),
            pltpu.VMEM((N_DEV, B, SKV_LOC, HD_LOC), jnp.float32),
            pltpu.VMEM((N_DEV, B, SKV_LOC, HD_LOC), jnp.float32),
            pltpu.VMEM((N_DEV, B * SQ, D_MODEL), jnp.float32),
            pltpu.VMEM((B * SQ, HD_LOC), jnp.float32),
            pltpu.SemaphoreType.DMA((N_DEV,)),
            pltpu.SemaphoreType.DMA((N_DEV,)),
            pltpu.SemaphoreType.DMA((N_DEV,)),
            pltpu.SemaphoreType.DMA((N_DEV,)),
            pltpu.SemaphoreType.DMA((N_DEV,)),
            pltpu.SemaphoreType.DMA((N_DEV,)),
        ],
        compiler_params=pltpu.CompilerParams(collective_id=0),
    )(x, Wq, K_ext, V_ext, Wo)


# device time: 137294 ns/iter; 1.1776x vs baseline; 1.1776x over previous
import jax
import jax.numpy as jnp
from jax import lax
from jax.experimental import pallas as pl
from jax.experimental.pallas import tpu as pltpu

N_DEV = 8
B, SQ, SKV, HQ, DH = 2, 256, 2048, 32, 64
H_LOC = HQ // N_DEV
SKV_LOC = SKV // N_DEV
HD_LOC = H_LOC * DH
D_MODEL = 512
ROWS = B * SQ


def kernel(x, Wq, K_ext, V_ext, Wo):
    def body(x_ref, wq_ref, k_ref, v_ref, wo_ref, out_ref,
             ksend, vsend, khbuf, vhbuf, acc, rs_stage, ctx2,
             ksend_sems, krecv_sems, vsend_sems, vrecv_sems,
             rs_send_sems, rs_recv_sems, ag_send_sems, ag_recv_sems):
        my = lax.axis_index("i")

        bar = pltpu.get_barrier_semaphore()
        for k in range(1, N_DEV):
            peer = lax.rem(my + k, N_DEV)
            pl.semaphore_signal(bar, inc=1, device_id=(peer,),
                                device_id_type=pl.DeviceIdType.MESH)
        pl.semaphore_wait(bar, N_DEV - 1)

        for i in range(N_DEV):
            ksend[i] = k_ref[:, :, H_LOC * i:H_LOC * (i + 1), :].reshape(
                B, SKV_LOC, HD_LOC)
            vsend[i] = v_ref[:, :, H_LOC * i:H_LOC * (i + 1), :].reshape(
                B, SKV_LOC, HD_LOC)

        khbuf[my] = ksend[my]
        vhbuf[my] = vsend[my]

        for k in range(1, N_DEV):
            dst = lax.rem(my + k, N_DEV)
            pltpu.make_async_remote_copy(
                src_ref=ksend.at[dst], dst_ref=khbuf.at[my],
                send_sem=ksend_sems.at[dst], recv_sem=krecv_sems.at[my],
                device_id=(dst,), device_id_type=pl.DeviceIdType.MESH,
            ).start()
            pltpu.make_async_remote_copy(
                src_ref=vsend.at[dst], dst_ref=vhbuf.at[my],
                send_sem=vsend_sems.at[dst], recv_sem=vrecv_sems.at[my],
                device_id=(dst,), device_id_type=pl.DeviceIdType.MESH,
            ).start()

        q2 = jnp.dot(x_ref[...].reshape(ROWS, D_MODEL), wq_ref[...],
                     preferred_element_type=jnp.float32)

        qb = lax.broadcasted_iota(jnp.int32, (SQ, SKV), 0) // 64
        kb = lax.broadcasted_iota(jnp.int32, (SQ, SKV), 1) // 64
        mask = (qb == kb) | (kb == 0) | ((qb + kb) % 3 == 0)

        for k in range(1, N_DEV):
            src = lax.rem(my + k, N_DEV)
            pltpu.make_async_remote_copy(
                src_ref=ksend.at[0], dst_ref=khbuf.at[src],
                send_sem=ksend_sems.at[0], recv_sem=krecv_sems.at[src],
                device_id=(src,), device_id_type=pl.DeviceIdType.MESH,
            ).wait_recv()
            pltpu.make_async_remote_copy(
                src_ref=vsend.at[0], dst_ref=vhbuf.at[src],
                send_sem=vsend_sems.at[0], recv_sem=vrecv_sems.at[src],
                device_id=(src,), device_id_type=pl.DeviceIdType.MESH,
            ).wait_recv()

        for b in range(B):
            for hl in range(H_LOC):
                q_bh = q2[b * SQ:(b + 1) * SQ, hl * DH:(hl + 1) * DH]
                kf = khbuf[:, b, :, hl * DH:(hl + 1) * DH].reshape(SKV, DH)
                s = lax.dot_general(
                    q_bh, kf, (((1,), (1,)), ((), ())),
                    preferred_element_type=jnp.float32) * 0.125
                s = jnp.where(mask, s, -1e9)
                m = jnp.max(s, axis=1, keepdims=True)
                w = jnp.exp(s - m)
                w = w / jnp.sum(w, axis=1, keepdims=True)
                vf = vhbuf[:, b, :, hl * DH:(hl + 1) * DH].reshape(SKV, DH)
                ctx2[b * SQ:(b + 1) * SQ, hl * DH:(hl + 1) * DH] = jnp.dot(
                    w, vf, preferred_element_type=jnp.float32)

        acc[...] = jnp.dot(ctx2[...], wo_ref[...],
                           preferred_element_type=jnp.float32)

        seg_base = jnp.int32(0)
        size = ROWS
        for idx, m in enumerate([4, 2, 1]):
            half = size // 2
            mybit = jnp.bitwise_and(my, m) // m
            partner = jnp.bitwise_xor(my, m)
            my_off = pl.multiple_of(seg_base + mybit * half, 64)
            other_off = pl.multiple_of(seg_base + (1 - mybit) * half, 64)
            rdma = pltpu.make_async_remote_copy(
                src_ref=acc.at[pl.ds(other_off, half), :],
                dst_ref=rs_stage.at[idx, pl.ds(0, half), :],
                send_sem=rs_send_sems.at[idx], recv_sem=rs_recv_sems.at[idx],
                device_id=(partner,), device_id_type=pl.DeviceIdType.MESH,
            )
            rdma.start()
            rdma.wait_recv()
            acc[pl.ds(my_off, half), :] = (
                acc[pl.ds(my_off, half), :] + rs_stage[idx, 0:half, :])
            seg_base = my_off
            size = half

        cur_base = pl.multiple_of(seg_base, 64)
        for idx, m in enumerate([1, 2, 4]):
            size_r = 64 * m
            partner = jnp.bitwise_xor(my, m)
            partner_base = pl.multiple_of(jnp.bitwise_xor(cur_base, 64 * m), 64)
            pltpu.make_async_remote_copy(
                src_ref=acc.at[pl.ds(cur_base, size_r), :],
                dst_ref=acc.at[pl.ds(cur_base, size_r), :],
                send_sem=ag_send_sems.at[idx], recv_sem=ag_recv_sems.at[idx],
                device_id=(partner,), device_id_type=pl.DeviceIdType.MESH,
            ).start()
            pltpu.make_async_remote_copy(
                src_ref=acc.at[pl.ds(cur_base, size_r), :],
                dst_ref=acc.at[pl.ds(partner_base, size_r), :],
                send_sem=ag_send_sems.at[idx], recv_sem=ag_recv_sems.at[idx],
                device_id=(partner,), device_id_type=pl.DeviceIdType.MESH,
            ).wait_recv()
            cur_base = pl.multiple_of(jnp.minimum(cur_base, partner_base), 64)

        out_ref[...] = acc[...].reshape(B, SQ, D_MODEL)

        for k in range(1, N_DEV):
            dst = lax.rem(my + k, N_DEV)
            pltpu.make_async_remote_copy(
                src_ref=ksend.at[dst], dst_ref=khbuf.at[0],
                send_sem=ksend_sems.at[dst], recv_sem=krecv_sems.at[0],
                device_id=(dst,), device_id_type=pl.DeviceIdType.MESH,
            ).wait_send()
            pltpu.make_async_remote_copy(
                src_ref=vsend.at[dst], dst_ref=vhbuf.at[0],
                send_sem=vsend_sems.at[dst], recv_sem=vrecv_sems.at[0],
                device_id=(dst,), device_id_type=pl.DeviceIdType.MESH,
            ).wait_send()
        for idx in range(3):
            half = 256 >> idx
            pltpu.make_async_remote_copy(
                src_ref=acc.at[pl.ds(0, half), :],
                dst_ref=rs_stage.at[idx, pl.ds(0, half), :],
                send_sem=rs_send_sems.at[idx], recv_sem=rs_recv_sems.at[idx],
                device_id=(my,), device_id_type=pl.DeviceIdType.MESH,
            ).wait_send()
            size_r = 64 << idx
            pltpu.make_async_remote_copy(
                src_ref=acc.at[pl.ds(0, size_r), :],
                dst_ref=acc.at[pl.ds(0, size_r), :],
                send_sem=ag_send_sems.at[idx], recv_sem=ag_recv_sems.at[idx],
                device_id=(my,), device_id_type=pl.DeviceIdType.MESH,
            ).wait_send()

    return pl.pallas_call(
        body,
        out_shape=jax.ShapeDtypeStruct((B, SQ, D_MODEL), jnp.float32),
        in_specs=[pl.BlockSpec(memory_space=pltpu.VMEM)] * 5,
        out_specs=pl.BlockSpec(memory_space=pltpu.VMEM),
        scratch_shapes=[
            pltpu.VMEM((N_DEV, B, SKV_LOC, HD_LOC), jnp.float32),
            pltpu.VMEM((N_DEV, B, SKV_LOC, HD_LOC), jnp.float32),
            pltpu.VMEM((N_DEV, B, SKV_LOC, HD_LOC), jnp.float32),
            pltpu.VMEM((N_DEV, B, SKV_LOC, HD_LOC), jnp.float32),
            pltpu.VMEM((ROWS, D_MODEL), jnp.float32),
            pltpu.VMEM((3, ROWS // 2, D_MODEL), jnp.float32),
            pltpu.VMEM((ROWS, HD_LOC), jnp.float32),
            pltpu.SemaphoreType.DMA((N_DEV,)),
            pltpu.SemaphoreType.DMA((N_DEV,)),
            pltpu.SemaphoreType.DMA((N_DEV,)),
            pltpu.SemaphoreType.DMA((N_DEV,)),
            pltpu.SemaphoreType.DMA((3,)),
            pltpu.SemaphoreType.DMA((3,)),
            pltpu.SemaphoreType.DMA((3,)),
            pltpu.SemaphoreType.DMA((3,)),
        ],
        compiler_params=pltpu.CompilerParams(collective_id=0),
    )(x, Wq, K_ext, V_ext, Wo)


# device time: 104407 ns/iter; 1.5486x vs baseline; 1.3150x over previous
import jax
import jax.numpy as jnp
from jax import lax
from jax.experimental import pallas as pl
from jax.experimental.pallas import tpu as pltpu

N_DEV = 8
B, SQ, SKV, HQ, DH = 2, 256, 2048, 32, 64
H_LOC = HQ // N_DEV
SKV_LOC = SKV // N_DEV
HD_LOC = H_LOC * DH
D_MODEL = 512
ROWS = B * SQ


def kernel(x, Wq, K_ext, V_ext, Wo):
    def body(x_ref, wq_ref, k_ref, v_ref, wo_ref, out_ref,
             ksend, vsend, khbuf, vhbuf, acc, rs_stage, ctx2,
             ksend_sems, krecv_sems, vsend_sems, vrecv_sems,
             rs_send_sems, rs_recv_sems, ag_send_sems, ag_recv_sems):
        my = lax.axis_index("i")

        bar = pltpu.get_barrier_semaphore()
        for k in range(1, N_DEV):
            peer = lax.rem(my + k, N_DEV)
            pl.semaphore_signal(bar, inc=1, device_id=(peer,),
                                device_id_type=pl.DeviceIdType.MESH)
        pl.semaphore_wait(bar, N_DEV - 1)

        for i in range(N_DEV):
            ksend[i] = k_ref[:, :, H_LOC * i:H_LOC * (i + 1), :].reshape(
                B, SKV_LOC, HD_LOC).astype(jnp.bfloat16)
            vsend[i] = v_ref[:, :, H_LOC * i:H_LOC * (i + 1), :].reshape(
                B, SKV_LOC, HD_LOC).astype(jnp.bfloat16)

        khbuf[my] = ksend[my]
        vhbuf[my] = vsend[my]

        for k in range(1, N_DEV):
            dst = lax.rem(my + k, N_DEV)
            pltpu.make_async_remote_copy(
                src_ref=ksend.at[dst], dst_ref=khbuf.at[my],
                send_sem=ksend_sems.at[dst], recv_sem=krecv_sems.at[my],
                device_id=(dst,), device_id_type=pl.DeviceIdType.MESH,
            ).start()
            pltpu.make_async_remote_copy(
                src_ref=vsend.at[dst], dst_ref=vhbuf.at[my],
                send_sem=vsend_sems.at[dst], recv_sem=vrecv_sems.at[my],
                device_id=(dst,), device_id_type=pl.DeviceIdType.MESH,
            ).start()

        q2 = jnp.dot(x_ref[...].reshape(ROWS, D_MODEL), wq_ref[...],
                     preferred_element_type=jnp.float32)

        qb = lax.broadcasted_iota(jnp.int32, (SQ, SKV), 0) // 64
        kb = lax.broadcasted_iota(jnp.int32, (SQ, SKV), 1) // 64
        mask = (qb == kb) | (kb == 0) | ((qb + kb) % 3 == 0)

        for k in range(1, N_DEV):
            src = lax.rem(my + k, N_DEV)
            pltpu.make_async_remote_copy(
                src_ref=ksend.at[0], dst_ref=khbuf.at[src],
                send_sem=ksend_sems.at[0], recv_sem=krecv_sems.at[src],
                device_id=(src,), device_id_type=pl.DeviceIdType.MESH,
            ).wait_recv()
            pltpu.make_async_remote_copy(
                src_ref=vsend.at[0], dst_ref=vhbuf.at[src],
                send_sem=vsend_sems.at[0], recv_sem=vrecv_sems.at[src],
                device_id=(src,), device_id_type=pl.DeviceIdType.MESH,
            ).wait_recv()

        for b in range(B):
            for hl in range(H_LOC):
                q_bh = q2[b * SQ:(b + 1) * SQ,
                          hl * DH:(hl + 1) * DH].astype(jnp.bfloat16)
                kf = khbuf[:, b, :, hl * DH:(hl + 1) * DH].reshape(SKV, DH)
                s = lax.dot_general(
                    q_bh, kf, (((1,), (1,)), ((), ())),
                    preferred_element_type=jnp.float32) * 0.125
                s = jnp.where(mask, s, -1e9)
                m = jnp.max(s, axis=1, keepdims=True)
                w = jnp.exp(s - m)
                w = (w / jnp.sum(w, axis=1, keepdims=True)).astype(jnp.bfloat16)
                vf = vhbuf[:, b, :, hl * DH:(hl + 1) * DH].reshape(SKV, DH)
                ctx2[b * SQ:(b + 1) * SQ, hl * DH:(hl + 1) * DH] = jnp.dot(
                    w, vf, preferred_element_type=jnp.float32)

        acc[...] = jnp.dot(ctx2[...], wo_ref[...],
                           preferred_element_type=jnp.float32)

        seg_base = jnp.int32(0)
        size = ROWS
        for idx, m in enumerate([4, 2, 1]):
            half = size // 2
            mybit = jnp.bitwise_and(my, m) // m
            partner = jnp.bitwise_xor(my, m)
            my_off = pl.multiple_of(seg_base + mybit * half, 64)
            other_off = pl.multiple_of(seg_base + (1 - mybit) * half, 64)
            rdma = pltpu.make_async_remote_copy(
                src_ref=acc.at[pl.ds(other_off, half), :],
                dst_ref=rs_stage.at[idx, pl.ds(0, half), :],
                send_sem=rs_send_sems.at[idx], recv_sem=rs_recv_sems.at[idx],
                device_id=(partner,), device_id_type=pl.DeviceIdType.MESH,
            )
            rdma.start()
            rdma.wait_recv()
            acc[pl.ds(my_off, half), :] = (
                acc[pl.ds(my_off, half), :] + rs_stage[idx, 0:half, :])
            seg_base = my_off
            size = half

        cur_base = pl.multiple_of(seg_base, 64)
        for idx, m in enumerate([1, 2, 4]):
            size_r = 64 * m
            partner = jnp.bitwise_xor(my, m)
            partner_base = pl.multiple_of(jnp.bitwise_xor(cur_base, 64 * m), 64)
            pltpu.make_async_remote_copy(
                src_ref=acc.at[pl.ds(cur_base, size_r), :],
                dst_ref=acc.at[pl.ds(cur_base, size_r), :],
                send_sem=ag_send_sems.at[idx], recv_sem=ag_recv_sems.at[idx],
                device_id=(partner,), device_id_type=pl.DeviceIdType.MESH,
            ).start()
            pltpu.make_async_remote_copy(
                src_ref=acc.at[pl.ds(cur_base, size_r), :],
                dst_ref=acc.at[pl.ds(partner_base, size_r), :],
                send_sem=ag_send_sems.at[idx], recv_sem=ag_recv_sems.at[idx],
                device_id=(partner,), device_id_type=pl.DeviceIdType.MESH,
            ).wait_recv()
            cur_base = pl.multiple_of(jnp.minimum(cur_base, partner_base), 64)

        out_ref[...] = acc[...].reshape(B, SQ, D_MODEL)

        for k in range(1, N_DEV):
            dst = lax.rem(my + k, N_DEV)
            pltpu.make_async_remote_copy(
                src_ref=ksend.at[dst], dst_ref=khbuf.at[0],
                send_sem=ksend_sems.at[dst], recv_sem=krecv_sems.at[0],
                device_id=(dst,), device_id_type=pl.DeviceIdType.MESH,
            ).wait_send()
            pltpu.make_async_remote_copy(
                src_ref=vsend.at[dst], dst_ref=vhbuf.at[0],
                send_sem=vsend_sems.at[dst], recv_sem=vrecv_sems.at[0],
                device_id=(dst,), device_id_type=pl.DeviceIdType.MESH,
            ).wait_send()
        for idx in range(3):
            half = 256 >> idx
            pltpu.make_async_remote_copy(
                src_ref=acc.at[pl.ds(0, half), :],
                dst_ref=rs_stage.at[idx, pl.ds(0, half), :],
                send_sem=rs_send_sems.at[idx], recv_sem=rs_recv_sems.at[idx],
                device_id=(my,), device_id_type=pl.DeviceIdType.MESH,
            ).wait_send()
            size_r = 64 << idx
            pltpu.make_async_remote_copy(
                src_ref=acc.at[pl.ds(0, size_r), :],
                dst_ref=acc.at[pl.ds(0, size_r), :],
                send_sem=ag_send_sems.at[idx], recv_sem=ag_recv_sems.at[idx],
                device_id=(my,), device_id_type=pl.DeviceIdType.MESH,
            ).wait_send()

    return pl.pallas_call(
        body,
        out_shape=jax.ShapeDtypeStruct((B, SQ, D_MODEL), jnp.float32),
        in_specs=[pl.BlockSpec(memory_space=pltpu.VMEM)] * 5,
        out_specs=pl.BlockSpec(memory_space=pltpu.VMEM),
        scratch_shapes=[
            pltpu.VMEM((N_DEV, B, SKV_LOC, HD_LOC), jnp.bfloat16),
            pltpu.VMEM((N_DEV, B, SKV_LOC, HD_LOC), jnp.bfloat16),
            pltpu.VMEM((N_DEV, B, SKV_LOC, HD_LOC), jnp.bfloat16),
            pltpu.VMEM((N_DEV, B, SKV_LOC, HD_LOC), jnp.bfloat16),
            pltpu.VMEM((ROWS, D_MODEL), jnp.float32),
            pltpu.VMEM((3, ROWS // 2, D_MODEL), jnp.float32),
            pltpu.VMEM((ROWS, HD_LOC), jnp.float32),
            pltpu.SemaphoreType.DMA((N_DEV,)),
            pltpu.SemaphoreType.DMA((N_DEV,)),
            pltpu.SemaphoreType.DMA((N_DEV,)),
            pltpu.SemaphoreType.DMA((N_DEV,)),
            pltpu.SemaphoreType.DMA((3,)),
            pltpu.SemaphoreType.DMA((3,)),
            pltpu.SemaphoreType.DMA((3,)),
            pltpu.SemaphoreType.DMA((3,)),
        ],
        compiler_params=pltpu.CompilerParams(collective_id=0),
    )(x, Wq, K_ext, V_ext, Wo)


# device time: 76142 ns/iter; 2.1235x vs baseline; 1.3712x over previous
import jax
import jax.numpy as jnp
from jax import lax
from jax.experimental import pallas as pl
from jax.experimental.pallas import tpu as pltpu

N_DEV = 8
B, SQ, SKV, HQ, DH = 2, 256, 2048, 32, 64
H_LOC = HQ // N_DEV
SKV_LOC = SKV // N_DEV
HD_LOC = H_LOC * DH
D_MODEL = 512
ROWS = B * SQ
ROW_LOC = ROWS // N_DEV
KV_BLOCKS_LOC = SKV_LOC // 64


def kernel(x, Wq, K_ext, V_ext, Wo):
    def body(x_ref, wq_ref, k_ref, v_ref, wo_ref, out_ref,
             ksend, vsend, khbuf, vhbuf, psend, precv, gbuf,
             ksend_sems, krecv_sems, vsend_sems, vrecv_sems,
             psend_sems, precv_sems, gsend_sems, grecv_sems):
        my = lax.axis_index("i")

        bar = pltpu.get_barrier_semaphore()
        for k in range(1, N_DEV):
            peer = lax.rem(my + k, N_DEV)
            pl.semaphore_signal(bar, inc=1, device_id=(peer,),
                                device_id_type=pl.DeviceIdType.MESH)
        pl.semaphore_wait(bar, N_DEV - 1)

        for i in range(N_DEV):
            ksend[i] = k_ref[:, :, H_LOC * i:H_LOC * (i + 1), :].reshape(
                B, SKV_LOC, HD_LOC).astype(jnp.bfloat16)
            vsend[i] = v_ref[:, :, H_LOC * i:H_LOC * (i + 1), :].reshape(
                B, SKV_LOC, HD_LOC).astype(jnp.bfloat16)

        khbuf[my] = ksend[my]
        vhbuf[my] = vsend[my]

        for k in range(1, N_DEV):
            dst = lax.rem(my + k, N_DEV)
            pltpu.make_async_remote_copy(
                src_ref=ksend.at[dst], dst_ref=khbuf.at[my],
                send_sem=ksend_sems.at[dst], recv_sem=krecv_sems.at[my],
                device_id=(dst,), device_id_type=pl.DeviceIdType.MESH,
            ).start()
            pltpu.make_async_remote_copy(
                src_ref=vsend.at[dst], dst_ref=vhbuf.at[my],
                send_sem=vsend_sems.at[dst], recv_sem=vrecv_sems.at[my],
                device_id=(dst,), device_id_type=pl.DeviceIdType.MESH,
            ).start()

        q2 = jnp.dot(x_ref[...].reshape(ROWS, D_MODEL), wq_ref[...],
                     preferred_element_type=jnp.float32).astype(jnp.bfloat16)

        qb_loc = lax.broadcasted_iota(jnp.int32, (SQ, SKV_LOC), 0) // 64
        kb_loc = lax.broadcasted_iota(jnp.int32, (SQ, SKV_LOC), 1) // 64

        M, S, C = {}, {}, {}

        def process(src, first):
            kb = kb_loc + src * KV_BLOCKS_LOC
            msk = (qb_loc == kb) | (kb == 0) | ((qb_loc + kb) % 3 == 0)
            for b in range(B):
                for hl in range(H_LOC):
                    q_bh = q2[b * SQ:(b + 1) * SQ, hl * DH:(hl + 1) * DH]
                    kf = khbuf[src, b, :, hl * DH:(hl + 1) * DH]
                    vf = vhbuf[src, b, :, hl * DH:(hl + 1) * DH]
                    s = lax.dot_general(
                        q_bh, kf, (((1,), (1,)), ((), ())),
                        preferred_element_type=jnp.float32) * 0.125
                    s = jnp.where(msk, s, -1e9)
                    mj = jnp.max(s, axis=1, keepdims=True)
                    if first:
                        mn = mj
                        p = jnp.exp(s - mn)
                        C[b, hl] = jnp.dot(p.astype(jnp.bfloat16), vf,
                                           preferred_element_type=jnp.float32)
                        S[b, hl] = jnp.sum(p, axis=1, keepdims=True)
                    else:
                        mn = jnp.maximum(M[b, hl], mj)
                        scale = jnp.exp(M[b, hl] - mn)
                        p = jnp.exp(s - mn)
                        C[b, hl] = C[b, hl] * scale + jnp.dot(
                            p.astype(jnp.bfloat16), vf,
                            preferred_element_type=jnp.float32)
                        S[b, hl] = S[b, hl] * scale + jnp.sum(
                            p, axis=1, keepdims=True)
                    M[b, hl] = mn

        process(my, first=True)
        for k in range(1, N_DEV):
            src = lax.rem(my - k + N_DEV, N_DEV)
            pltpu.make_async_remote_copy(
                src_ref=ksend.at[0], dst_ref=khbuf.at[src],
                send_sem=ksend_sems.at[0], recv_sem=krecv_sems.at[src],
                device_id=(src,), device_id_type=pl.DeviceIdType.MESH,
            ).wait_recv()
            pltpu.make_async_remote_copy(
                src_ref=vsend.at[0], dst_ref=vhbuf.at[src],
                send_sem=vsend_sems.at[0], recv_sem=vrecv_sems.at[src],
                device_id=(src,), device_id_type=pl.DeviceIdType.MESH,
            ).wait_recv()
            process(src, first=False)

        ctx = jnp.concatenate(
            [jnp.concatenate([C[b, hl] / S[b, hl] for hl in range(H_LOC)],
                             axis=1) for b in range(B)],
            axis=0)

        part = jnp.dot(ctx, wo_ref[...],
                       preferred_element_type=jnp.float32)

        for i in range(N_DEV):
            psend[i] = part[ROW_LOC * i:ROW_LOC * (i + 1), :].astype(
                jnp.bfloat16)
        precv[my] = psend[my]
        for k in range(1, N_DEV):
            dst = lax.rem(my + k, N_DEV)
            pltpu.make_async_remote_copy(
                src_ref=psend.at[dst], dst_ref=precv.at[my],
                send_sem=psend_sems.at[dst], recv_sem=precv_sems.at[my],
                device_id=(dst,), device_id_type=pl.DeviceIdType.MESH,
            ).start()
        for k in range(1, N_DEV):
            src = lax.rem(my + k, N_DEV)
            pltpu.make_async_remote_copy(
                src_ref=psend.at[0], dst_ref=precv.at[src],
                send_sem=psend_sems.at[0], recv_sem=precv_sems.at[src],
                device_id=(src,), device_id_type=pl.DeviceIdType.MESH,
            ).wait_recv()
        red = jnp.sum(precv[...].astype(jnp.float32), axis=0)

        gbuf[my] = red.astype(jnp.bfloat16)
        for k in range(1, N_DEV):
            dst = lax.rem(my + k, N_DEV)
            pltpu.make_async_remote_copy(
                src_ref=gbuf.at[my], dst_ref=gbuf.at[my],
                send_sem=gsend_sems.at[dst], recv_sem=grecv_sems.at[my],
                device_id=(dst,), device_id_type=pl.DeviceIdType.MESH,
            ).start()
        for k in range(1, N_DEV):
            src = lax.rem(my + k, N_DEV)
            pltpu.make_async_remote_copy(
                src_ref=gbuf.at[0], dst_ref=gbuf.at[src],
                send_sem=gsend_sems.at[0], recv_sem=grecv_sems.at[src],
                device_id=(src,), device_id_type=pl.DeviceIdType.MESH,
            ).wait_recv()

        out_ref[...] = gbuf[...].astype(jnp.float32).reshape(B, SQ, D_MODEL)

        for k in range(1, N_DEV):
            dst = lax.rem(my + k, N_DEV)
            pltpu.make_async_remote_copy(
                src_ref=ksend.at[dst], dst_ref=khbuf.at[0],
                send_sem=ksend_sems.at[dst], recv_sem=krecv_sems.at[0],
                device_id=(dst,), device_id_type=pl.DeviceIdType.MESH,
            ).wait_send()
            pltpu.make_async_remote_copy(
                src_ref=vsend.at[dst], dst_ref=vhbuf.at[0],
                send_sem=vsend_sems.at[dst], recv_sem=vrecv_sems.at[0],
                device_id=(dst,), device_id_type=pl.DeviceIdType.MESH,
            ).wait_send()
            pltpu.make_async_remote_copy(
                src_ref=psend.at[dst], dst_ref=precv.at[0],
                send_sem=psend_sems.at[dst], recv_sem=precv_sems.at[0],
                device_id=(dst,), device_id_type=pl.DeviceIdType.MESH,
            ).wait_send()
            pltpu.make_async_remote_copy(
                src_ref=gbuf.at[my], dst_ref=gbuf.at[0],
                send_sem=gsend_sems.at[dst], recv_sem=grecv_sems.at[0],
                device_id=(dst,), device_id_type=pl.DeviceIdType.MESH,
            ).wait_send()

    return pl.pallas_call(
        body,
        out_shape=jax.ShapeDtypeStruct((B, SQ, D_MODEL), jnp.float32),
        in_specs=[pl.BlockSpec(memory_space=pltpu.VMEM)] * 5,
        out_specs=pl.BlockSpec(memory_space=pltpu.VMEM),
        scratch_shapes=[
            pltpu.VMEM((N_DEV, B, SKV_LOC, HD_LOC), jnp.bfloat16),
            pltpu.VMEM((N_DEV, B, SKV_LOC, HD_LOC), jnp.bfloat16),
            pltpu.VMEM((N_DEV, B, SKV_LOC, HD_LOC), jnp.bfloat16),
            pltpu.VMEM((N_DEV, B, SKV_LOC, HD_LOC), jnp.bfloat16),
            pltpu.VMEM((N_DEV, ROW_LOC, D_MODEL), jnp.bfloat16),
            pltpu.VMEM((N_DEV, ROW_LOC, D_MODEL), jnp.bfloat16),
            pltpu.VMEM((N_DEV, ROW_LOC, D_MODEL), jnp.bfloat16),
            pltpu.SemaphoreType.DMA((N_DEV,)),
            pltpu.SemaphoreType.DMA((N_DEV,)),
            pltpu.SemaphoreType.DMA((N_DEV,)),
            pltpu.SemaphoreType.DMA((N_DEV,)),
            pltpu.SemaphoreType.DMA((N_DEV,)),
            pltpu.SemaphoreType.DMA((N_DEV,)),
            pltpu.SemaphoreType.DMA((N_DEV,)),
            pltpu.SemaphoreType.DMA((N_DEV,)),
        ],
        compiler_params=pltpu.CompilerParams(collective_id=0),
    )(x, Wq, K_ext, V_ext, Wo)


# device time: 59828 ns/iter; 2.7025x vs baseline; 1.2727x over previous
import jax
import jax.numpy as jnp
from jax import lax
from jax.experimental import pallas as pl
from jax.experimental.pallas import tpu as pltpu

N_DEV = 8
B, SQ, SKV, HQ, DH = 2, 256, 2048, 32, 64
H_LOC = HQ // N_DEV
SKV_LOC = SKV // N_DEV
HD_LOC = H_LOC * DH
D_MODEL = 512
ROWS = B * SQ
ROW_LOC = ROWS // N_DEV
KV_BLOCKS_LOC = SKV_LOC // 64


def kernel(x, Wq, K_ext, V_ext, Wo):
    Kt = jnp.transpose(
        K_ext.reshape(B, SKV_LOC, N_DEV, HD_LOC), (2, 0, 1, 3)
    ).astype(jnp.bfloat16)
    Vt = jnp.transpose(
        V_ext.reshape(B, SKV_LOC, N_DEV, HD_LOC), (2, 0, 1, 3)
    ).astype(jnp.bfloat16)

    def body(x_ref, wq_ref, kt_ref, vt_ref, wo_ref, out_ref,
             khbuf, vhbuf, psend, precv, gbuf,
             ksend_sems, krecv_sems, vsend_sems, vrecv_sems,
             psend_sems, precv_sems, gsend_sems, grecv_sems):
        my = lax.axis_index("i")

        bar = pltpu.get_barrier_semaphore()
        for k in range(1, N_DEV):
            peer = lax.rem(my + k, N_DEV)
            pl.semaphore_signal(bar, inc=1, device_id=(peer,),
                                device_id_type=pl.DeviceIdType.MESH)
        pl.semaphore_wait(bar, N_DEV - 1)

        for k in range(1, N_DEV):
            dst = lax.rem(my + k, N_DEV)
            pltpu.make_async_remote_copy(
                src_ref=kt_ref.at[dst], dst_ref=khbuf.at[my],
                send_sem=ksend_sems.at[dst], recv_sem=krecv_sems.at[my],
                device_id=(dst,), device_id_type=pl.DeviceIdType.MESH,
            ).start()
            pltpu.make_async_remote_copy(
                src_ref=vt_ref.at[dst], dst_ref=vhbuf.at[my],
                send_sem=vsend_sems.at[dst], recv_sem=vrecv_sems.at[my],
                device_id=(dst,), device_id_type=pl.DeviceIdType.MESH,
            ).start()

        q2 = jnp.dot(x_ref[...].reshape(ROWS, D_MODEL), wq_ref[...],
                     preferred_element_type=jnp.float32).astype(jnp.bfloat16)

        qb_loc = lax.broadcasted_iota(jnp.int32, (SQ, SKV_LOC), 0) // 64
        kb_loc = lax.broadcasted_iota(jnp.int32, (SQ, SKV_LOC), 1) // 64

        M, S, C = {}, {}, {}

        def process(src, first, kref, vref):
            kb = kb_loc + src * KV_BLOCKS_LOC
            msk = (qb_loc == kb) | (kb == 0) | ((qb_loc + kb) % 3 == 0)
            for b in range(B):
                for hl in range(H_LOC):
                    q_bh = q2[b * SQ:(b + 1) * SQ, hl * DH:(hl + 1) * DH]
                    kf = kref[src, b, :, hl * DH:(hl + 1) * DH]
                    vf = vref[src, b, :, hl * DH:(hl + 1) * DH]
                    s = lax.dot_general(
                        q_bh, kf, (((1,), (1,)), ((), ())),
                        preferred_element_type=jnp.float32) * 0.125
                    s = jnp.where(msk, s, -1e9)
                    mj = jnp.max(s, axis=1, keepdims=True)
                    if first:
                        mn = mj
                        p = jnp.exp(s - mn)
                        C[b, hl] = jnp.dot(p.astype(jnp.bfloat16), vf,
                                           preferred_element_type=jnp.float32)
                        S[b, hl] = jnp.sum(p, axis=1, keepdims=True)
                    else:
                        mn = jnp.maximum(M[b, hl], mj)
                        scale = jnp.exp(M[b, hl] - mn)
                        p = jnp.exp(s - mn)
                        C[b, hl] = C[b, hl] * scale + jnp.dot(
                            p.astype(jnp.bfloat16), vf,
                            preferred_element_type=jnp.float32)
                        S[b, hl] = S[b, hl] * scale + jnp.sum(
                            p, axis=1, keepdims=True)
                    M[b, hl] = mn

        process(my, True, kt_ref, vt_ref)
        for k in range(1, N_DEV):
            src = lax.rem(my - k + N_DEV, N_DEV)
            pltpu.make_async_remote_copy(
                src_ref=kt_ref.at[0], dst_ref=khbuf.at[src],
                send_sem=ksend_sems.at[0], recv_sem=krecv_sems.at[src],
                device_id=(src,), device_id_type=pl.DeviceIdType.MESH,
            ).wait_recv()
            pltpu.make_async_remote_copy(
                src_ref=vt_ref.at[0], dst_ref=vhbuf.at[src],
                send_sem=vsend_sems.at[0], recv_sem=vrecv_sems.at[src],
                device_id=(src,), device_id_type=pl.DeviceIdType.MESH,
            ).wait_recv()
            process(src, False, khbuf, vhbuf)

        ctx = jnp.concatenate(
            [jnp.concatenate([C[b, hl] / S[b, hl] for hl in range(H_LOC)],
                             axis=1) for b in range(B)],
            axis=0)

        part = jnp.dot(ctx, wo_ref[...],
                       preferred_element_type=jnp.float32)

        for i in range(N_DEV):
            psend[i] = part[ROW_LOC * i:ROW_LOC * (i + 1), :].astype(
                jnp.bfloat16)
        precv[my] = psend[my]
        for k in range(1, N_DEV):
            dst = lax.rem(my + k, N_DEV)
            pltpu.make_async_remote_copy(
                src_ref=psend.at[dst], dst_ref=precv.at[my],
                send_sem=psend_sems.at[dst], recv_sem=precv_sems.at[my],
                device_id=(dst,), device_id_type=pl.DeviceIdType.MESH,
            ).start()
        for k in range(1, N_DEV):
            src = lax.rem(my + k, N_DEV)
            pltpu.make_async_remote_copy(
                src_ref=psend.at[0], dst_ref=precv.at[src],
                send_sem=psend_sems.at[0], recv_sem=precv_sems.at[src],
                device_id=(src,), device_id_type=pl.DeviceIdType.MESH,
            ).wait_recv()
        red = jnp.sum(precv[...].astype(jnp.float32), axis=0)

        gbuf[my] = red.astype(jnp.bfloat16)
        for k in range(1, N_DEV):
            dst = lax.rem(my + k, N_DEV)
            pltpu.make_async_remote_copy(
                src_ref=gbuf.at[my], dst_ref=gbuf.at[my],
                send_sem=gsend_sems.at[dst], recv_sem=grecv_sems.at[my],
                device_id=(dst,), device_id_type=pl.DeviceIdType.MESH,
            ).start()
        for k in range(1, N_DEV):
            src = lax.rem(my + k, N_DEV)
            pltpu.make_async_remote_copy(
                src_ref=gbuf.at[0], dst_ref=gbuf.at[src],
                send_sem=gsend_sems.at[0], recv_sem=grecv_sems.at[src],
                device_id=(src,), device_id_type=pl.DeviceIdType.MESH,
            ).wait_recv()

        out_ref[...] = gbuf[...].astype(jnp.float32).reshape(B, SQ, D_MODEL)

        for k in range(1, N_DEV):
            dst = lax.rem(my + k, N_DEV)
            pltpu.make_async_remote_copy(
                src_ref=kt_ref.at[dst], dst_ref=khbuf.at[0],
                send_sem=ksend_sems.at[dst], recv_sem=krecv_sems.at[0],
                device_id=(dst,), device_id_type=pl.DeviceIdType.MESH,
            ).wait_send()
            pltpu.make_async_remote_copy(
                src_ref=vt_ref.at[dst], dst_ref=vhbuf.at[0],
                send_sem=vsend_sems.at[dst], recv_sem=vrecv_sems.at[0],
                device_id=(dst,), device_id_type=pl.DeviceIdType.MESH,
            ).wait_send()
            pltpu.make_async_remote_copy(
                src_ref=psend.at[dst], dst_ref=precv.at[0],
                send_sem=psend_sems.at[dst], recv_sem=precv_sems.at[0],
                device_id=(dst,), device_id_type=pl.DeviceIdType.MESH,
            ).wait_send()
            pltpu.make_async_remote_copy(
                src_ref=gbuf.at[my], dst_ref=gbuf.at[0],
                send_sem=gsend_sems.at[dst], recv_sem=grecv_sems.at[0],
                device_id=(dst,), device_id_type=pl.DeviceIdType.MESH,
            ).wait_send()

    return pl.pallas_call(
        body,
        out_shape=jax.ShapeDtypeStruct((B, SQ, D_MODEL), jnp.float32),
        in_specs=[pl.BlockSpec(memory_space=pltpu.VMEM)] * 5,
        out_specs=pl.BlockSpec(memory_space=pltpu.VMEM),
        scratch_shapes=[
            pltpu.VMEM((N_DEV, B, SKV_LOC, HD_LOC), jnp.bfloat16),
            pltpu.VMEM((N_DEV, B, SKV_LOC, HD_LOC), jnp.bfloat16),
            pltpu.VMEM((N_DEV, ROW_LOC, D_MODEL), jnp.bfloat16),
            pltpu.VMEM((N_DEV, ROW_LOC, D_MODEL), jnp.bfloat16),
            pltpu.VMEM((N_DEV, ROW_LOC, D_MODEL), jnp.bfloat16),
            pltpu.SemaphoreType.DMA((N_DEV,)),
            pltpu.SemaphoreType.DMA((N_DEV,)),
            pltpu.SemaphoreType.DMA((N_DEV,)),
            pltpu.SemaphoreType.DMA((N_DEV,)),
            pltpu.SemaphoreType.DMA((N_DEV,)),
            pltpu.SemaphoreType.DMA((N_DEV,)),
            pltpu.SemaphoreType.DMA((N_DEV,)),
            pltpu.SemaphoreType.DMA((N_DEV,)),
        ],
        compiler_params=pltpu.CompilerParams(collective_id=0),
    )(x, Wq, Kt, Vt, Wo)
